# baseline (speedup 1.0000x reference)
import functools

import jax
import jax.numpy as jnp
from jax import lax
from jax.experimental import pallas as pl
from jax.experimental.pallas import tpu as pltpu
from jax.experimental.pallas import tpu_sc as plsc

N = 10000
E = 320000
NC, NS, L = 2, 16, 16
NW = NC * NS
EPW = E // NW
ROWS_PER_TILE = N // NS


def _zero_blocks(chunk):
    out, r = [], 0
    while r + chunk <= ROWS_PER_TILE:
        out.append((r, chunk))
        r += chunk
    if r < ROWS_PER_TILE:
        out.append((r, ROWS_PER_TILE - r))
    return out


def _make_segsum(D, CHUNK, NCHUNKS):
    JV = D // L
    _BLOCKS = _zero_blocks(CHUNK)
    mesh = plsc.VectorSubcoreMesh(core_axis_name="c", subcore_axis_name="s")

    @functools.partial(
        pl.kernel,
        out_type=jax.ShapeDtypeStruct((NC, N, D), jnp.float32),
        mesh=mesh,
        compiler_params=pltpu.CompilerParams(use_tc_tiling_on_sc=False),
        scratch_types=[
            pltpu.VMEM((8, 3, CHUNK), jnp.int32),
            pltpu.VMEM((4, CHUNK, D), jnp.float32),
            pltpu.VMEM_SHARED((N, D), jnp.float32),
            [pltpu.SemaphoreType.DMA] * 4,
            [pltpu.SemaphoreType.DMA] * 4,
            [pltpu.SemaphoreType.DMA] * 8,
        ],
    )
    def seg(h_hbm, comb_hbm, out_hbm,
            ring_v, rows_v, acc_sh, gsem, ssem, csem):
        c = lax.axis_index("c")
        s = lax.axis_index("s")
        wid = s * NC + c

        def start_comb(i, cs):
            pltpu.async_copy(comb_hbm.at[wid, i], ring_v.at[cs], csem[cs])

        def wait_comb(i, cs):
            pltpu.make_async_copy(comb_hbm.at[wid, i], ring_v.at[cs],
                                  csem[cs]).wait()

        def start_gather(cs, rs):
            pltpu.async_copy(h_hbm.at[ring_v.at[cs, 0]], rows_v.at[rs],
                             gsem[rs])

        def wait_gather(cs, rs):
            pltpu.make_async_copy(h_hbm.at[ring_v.at[cs, 0]], rows_v.at[rs],
                                  gsem[rs]).wait()

        def start_scat(cs, rs):
            pltpu.async_copy(rows_v.at[rs], acc_sh.at[ring_v.at[cs, 1]],
                             ssem[rs], add=True)

        def wait_scat(cs, rs):
            pltpu.make_async_copy(rows_v.at[rs], acc_sh.at[ring_v.at[cs, 1]],
                                  ssem[rs]).wait()

        def scale(i, cs, rs):
            def group_body(g, _):
                av = lax.bitcast_convert_type(
                    ring_v[cs, 2, pl.ds(g * L, L)], jnp.float32)
                for e16 in range(L):
                    a = av[e16]
                    rr = g * L + e16
                    for j in range(JV):
                        sl = pl.ds(j * L, L)
                        rows_v[rs, rr, sl] = rows_v[rs, rr, sl] * a
                return 0
            lax.fori_loop(0, CHUNK // L, group_body, 0)

        for k in range(6):
            start_comb(k, k)
        wait_comb(0, 0)
        start_gather(0, 0)
        wait_comb(1, 1)
        start_gather(1, 1)

        def zero_row(r, _):
            for j in range(JV):
                rows_v[3, r, pl.ds(j * L, L)] = jnp.zeros((L,), jnp.float32)
            return 0
        lax.fori_loop(0, CHUNK, zero_row, 0)
        for r0, nr in _BLOCKS:
            pltpu.sync_copy(
                rows_v.at[3, pl.ds(0, nr)],
                acc_sh.at[pl.ds(s * ROWS_PER_TILE + r0, nr)])
        plsc.subcore_barrier()

        def octet_body(k, _):
            for p in range(8):
                i = 8 * k + p
                rs = p & 3

                @pl.when(i <= NCHUNKS - 1)
                def _():
                    wait_gather(p, rs)

                @pl.when(jnp.logical_and(i >= 2, i <= NCHUNKS + 1))
                def _():
                    wait_scat((p + 6) % 8, (p + 2) % 4)

                @pl.when(i <= NCHUNKS - 7)
                def _():
                    start_comb(i + 6, (p + 6) % 8)

                @pl.when(i <= NCHUNKS - 3)
                def _():
                    wait_comb(i + 2, (p + 2) % 8)
                    start_gather((p + 2) % 8, (p + 2) % 4)

                @pl.when(i <= NCHUNKS - 1)
                def _():
                    scale(i, p, rs)
                    start_scat(p, rs)
            return 0
        lax.fori_loop(0, (NCHUNKS + 9) // 8, octet_body, 0)

        plsc.subcore_barrier()

        a0 = s * ROWS_PER_TILE
        pltpu.sync_copy(acc_sh.at[pl.ds(a0, ROWS_PER_TILE)],
                        out_hbm.at[c, pl.ds(a0, ROWS_PER_TILE)])

    return seg


_segsum128 = _make_segsum(128, 80, 125)
_segsum64 = _make_segsum(64, 128, 79)

_DENSE_BLK = 2000


def _dense_body(h_ref, p0_ref, p1_ref, w1_ref, b1_ref, w2_ref, b2_ref, o_ref):
    h = h_ref[...]
    nh = p0_ref[0] + p1_ref[0]
    z1 = jnp.dot(h + nh, w1_ref[...],
                 preferred_element_type=jnp.float32) + b1_ref[...]
    z2 = jnp.dot(h * nh, w2_ref[...],
                 preferred_element_type=jnp.float32) + b2_ref[...]
    o_ref[...] = (jnp.where(z1 >= 0, z1, 0.01 * z1)
                  + jnp.where(z2 >= 0, z2, 0.01 * z2))


def _dense(h, parts, w1, b1, w2, b2):
    din = h.shape[1]
    dout = w1.shape[1]
    b1 = b1.reshape(1, dout)
    b2 = b2.reshape(1, dout)
    return pl.pallas_call(
        _dense_body,
        grid=(N // _DENSE_BLK,),
        in_specs=[
            pl.BlockSpec((_DENSE_BLK, din), lambda i: (i, 0)),
            pl.BlockSpec((1, _DENSE_BLK, din), lambda i: (0, i, 0)),
            pl.BlockSpec((1, _DENSE_BLK, din), lambda i: (1, i, 0)),
            pl.BlockSpec((din, dout), lambda i: (0, 0)),
            pl.BlockSpec((1, dout), lambda i: (0, 0)),
            pl.BlockSpec((din, dout), lambda i: (0, 0)),
            pl.BlockSpec((1, dout), lambda i: (0, 0)),
        ],
        out_specs=pl.BlockSpec((_DENSE_BLK, dout), lambda i: (i, 0)),
        out_shape=jax.ShapeDtypeStruct((N, dout), jnp.float32),
    )(h, parts, parts, w1, b1, w2, b2)


def _mk_comb(edge_index, att, chunk, nchunks):
    pad = nchunks * chunk - EPW

    def prep(a):
        a2 = a.reshape(NW, EPW)
        if pad:
            a2 = jnp.pad(a2, ((0, 0), (0, pad)))
        return a2.reshape(NW, nchunks, 1, chunk)

    att_i = lax.bitcast_convert_type(att, jnp.int32)
    return jnp.concatenate(
        [prep(edge_index[0]), prep(edge_index[1]), prep(att_i)], axis=2)


def kernel(x, edge_index, att, W1_0, b1_0, W2_0, b2_0, W1_1, b1_1, W2_1, b2_1):
    comb1 = _mk_comb(edge_index, att, 80, 125)
    comb2 = _mk_comb(edge_index, att, 128, 79)
    parts1 = _segsum128(x, comb1)
    h1 = _dense(x, parts1, W1_0, b1_0, W2_0, b2_0)
    parts2 = _segsum64(h1, comb2)
    out = _dense(h1, parts2, W1_1, b1_1, W2_1, b2_1)
    return out
